# Initial kernel scaffold; baseline (speedup 1.0000x reference)
import jax
import jax.numpy as jnp
from jax.experimental import pallas as pl


def kernel(x, edge_index, edges, rel_embed, Wq, bq, Wk, Wv, Wo, bo, ln1_g, ln1_b, W1, b1, W2, b2, ln2_g, ln2_b):
    raise NotImplementedError("write your pallas kernel here")


# speedup vs baseline: 25.4223x; 25.4223x over previous
import functools
import numpy as np
import jax
import jax.numpy as jnp
from jax import lax
from jax.experimental import pallas as pl
from jax.experimental.pallas import tpu as pltpu
from jax.experimental.pallas import tpu_sc as plsc

N = 10000
E = 320000
D = 128
H = 8
DK = 16
L = 2
R = 50
FF = 4 * D

NC = 2
NS = 16
NW = NC * NS
EPW = E // NW
C = 40
NCHUNK = EPW // C
RPT = 624
REM = N - NS * RPT
ZR = 48


_BLK = 1000


def _qkv_body(x_ref, w_ref, bq_ref, q_ref, k_ref, v_ref):
    x = x_ref[...]
    q_ref[...] = jnp.dot(x, w_ref[0], preferred_element_type=jnp.float32) + bq_ref[...]
    k_ref[...] = jnp.dot(x, w_ref[1], preferred_element_type=jnp.float32)
    v_ref[...] = jnp.dot(x, w_ref[2], preferred_element_type=jnp.float32)


def _tc_qkv(x, wstack, bq):
    return pl.pallas_call(
        _qkv_body,
        grid=(N // _BLK,),
        in_specs=[
            pl.BlockSpec((_BLK, D), lambda i: (i, 0)),
            pl.BlockSpec((3, D, D), lambda i: (0, 0, 0)),
            pl.BlockSpec((1, D), lambda i: (0, 0)),
        ],
        out_specs=[
            pl.BlockSpec((_BLK, D), lambda i: (i, 0)),
            pl.BlockSpec((_BLK, D), lambda i: (i, 0)),
            pl.BlockSpec((_BLK, D), lambda i: (i, 0)),
        ],
        out_shape=[
            jax.ShapeDtypeStruct((N, D), jnp.float32),
            jax.ShapeDtypeStruct((N, D), jnp.float32),
            jax.ShapeDtypeStruct((N, D), jnp.float32),
        ],
    )(x, wstack, bq)


def _sc_edge_body(q_hbm, k_hbm, v_hbm, rel_hbm, src_hbm, dst_hbm, rid_hbm,
                  outwv_hbm, outz_hbm,
                  accwv, accz, sidx, didx, ridx,
                  krows, qrows, vrows, relrows, msg, zbuf, zwv0, zz0):
    c = lax.axis_index("c")
    s = lax.axis_index("s")
    wid = c * NS + s

    zero16 = jnp.zeros((16,), jnp.float32)

    def _zrow(i, carry):
        for j in range(D // 16):
            zwv0[i, pl.ds(j * 16, 16)] = zero16
        zz0[i, :] = zero16
        return carry

    lax.fori_loop(0, ZR, _zrow, 0)
    for rep in range(RPT // ZR):
        base = s * RPT + rep * ZR
        pltpu.sync_copy(zwv0, accwv.at[pl.ds(base, ZR)])
        pltpu.sync_copy(zz0, accz.at[pl.ds(base, ZR)])

    @pl.when(s == NS - 1)
    def _zero_tail():
        pltpu.sync_copy(zwv0.at[pl.ds(0, REM)], accwv.at[pl.ds(NS * RPT, REM)])
        pltpu.sync_copy(zz0.at[pl.ds(0, REM)], accz.at[pl.ds(NS * RPT, REM)])

    plsc.subcore_barrier()

    iota = lax.iota(jnp.int32, 16)
    inv_scale = 1.0 / float(np.sqrt(DK))
    shuf = [(iota + sh) & 15 for sh in (8, 4, 2, 1)]

    dnums = lax.GatherDimensionNumbers(
        offset_dims=(), collapsed_slice_dims=(0,), start_index_map=(0,))

    def _allsum(t):
        for sx in shuf:
            t = t + lax.gather(t, sx[:, None], dnums, slice_sizes=(1,),
                               mode=lax.GatherScatterMode.PROMISE_IN_BOUNDS)
        return t

    def _chunk(t, carry):
        ebase = wid * EPW + t * C
        pltpu.sync_copy(src_hbm.at[pl.ds(ebase, C)], sidx)
        pltpu.sync_copy(dst_hbm.at[pl.ds(ebase, C)], didx)
        pltpu.sync_copy(rid_hbm.at[pl.ds(ebase, C)], ridx)
        pltpu.sync_copy(k_hbm.at[sidx], krows)
        pltpu.sync_copy(q_hbm.at[didx], qrows)
        pltpu.sync_copy(v_hbm.at[sidx], vrows)
        pltpu.sync_copy(rel_hbm.at[ridx], relrows)

        def _edge(i, ecarry):
            rel = relrows[i, :]
            zv = zero16
            for h in range(H):
                sl = pl.ds(h * 16, 16)
                kh = krows[i, sl]
                qh = qrows[i, sl]
                vh = vrows[i, sl]
                sb = _allsum((kh + rel) * qh) * inv_scale
                se = jnp.exp(jnp.clip(sb, -10.0, 10.0))
                msg[i, sl] = (vh + rel) * se
                zv = jnp.where(iota == h, se, zv)
            zbuf[i, :] = zv
            return ecarry

        lax.fori_loop(0, C, _edge, 0)
        pltpu.sync_copy(msg, accwv.at[didx], add=True)
        pltpu.sync_copy(zbuf, accz.at[didx], add=True)
        return carry

    lax.fori_loop(0, NCHUNK, _chunk, 0)
    plsc.subcore_barrier()

    row0 = s * RPT
    pltpu.sync_copy(accwv.at[pl.ds(row0, RPT)], outwv_hbm.at[c, pl.ds(row0, RPT)])
    pltpu.sync_copy(accz.at[pl.ds(row0, RPT)], outz_hbm.at[c, pl.ds(row0, RPT)])

    @pl.when(s == NS - 1)
    def _copy_tail():
        pltpu.sync_copy(accwv.at[pl.ds(NS * RPT, REM)],
                        outwv_hbm.at[c, pl.ds(NS * RPT, REM)])
        pltpu.sync_copy(accz.at[pl.ds(NS * RPT, REM)],
                        outz_hbm.at[c, pl.ds(NS * RPT, REM)])


_sc_edge = functools.partial(
    pl.kernel,
    _sc_edge_body,
    out_type=(
        jax.ShapeDtypeStruct((NC, N, D), jnp.float32),
        jax.ShapeDtypeStruct((NC, N, 16), jnp.float32),
    ),
    mesh=plsc.VectorSubcoreMesh(core_axis_name="c", subcore_axis_name="s"),
    compiler_params=pltpu.CompilerParams(use_tc_tiling_on_sc=False),
    scratch_types=[
        pltpu.VMEM_SHARED((N, D), jnp.float32),
        pltpu.VMEM_SHARED((N, 16), jnp.float32),
        pltpu.VMEM((C,), jnp.int32),
        pltpu.VMEM((C,), jnp.int32),
        pltpu.VMEM((C,), jnp.int32),
        pltpu.VMEM((C, D), jnp.float32),
        pltpu.VMEM((C, D), jnp.float32),
        pltpu.VMEM((C, D), jnp.float32),
        pltpu.VMEM((C, 16), jnp.float32),
        pltpu.VMEM((C, D), jnp.float32),
        pltpu.VMEM((C, 16), jnp.float32),
        pltpu.VMEM((ZR, D), jnp.float32),
        pltpu.VMEM((ZR, 16), jnp.float32),
    ],
)()


def _ln(a, g, b, eps=1e-5):
    m = jnp.mean(a, axis=-1, keepdims=True)
    v = jnp.mean((a - m) ** 2, axis=-1, keepdims=True)
    return g * (a - m) / jnp.sqrt(v + eps) + b


def _post_body(awv_ref, az_ref, x_ref, e2_ref, wo_ref, bo_ref, g1_ref, b1n_ref,
               w1_ref, b1_ref, w2_ref, b2_ref, g2_ref, b2n_ref, out_ref):
    wv = awv_ref[0] + awv_ref[1]
    zh = az_ref[0] + az_ref[1]
    z128 = jnp.dot(zh, e2_ref[...], preferred_element_type=jnp.float32)
    o = wv / (z128 + 1e-12)
    a = x_ref[...] + jnp.dot(o, wo_ref[...], preferred_element_type=jnp.float32) + bo_ref[...]
    x1 = _ln(a, g1_ref[...], b1n_ref[...])
    hmid = jnp.maximum(jnp.dot(x1, w1_ref[...], preferred_element_type=jnp.float32) + b1_ref[...], 0.0)
    hout = jnp.dot(hmid, w2_ref[...], preferred_element_type=jnp.float32) + b2_ref[...]
    out_ref[...] = _ln(x1 + hout, g2_ref[...], b2n_ref[...])


def _tc_post(awv, az, x, e2, wo, bo, g1, b1n, w1, b1, w2, b2, g2, b2n):
    return pl.pallas_call(
        _post_body,
        grid=(N // _BLK,),
        in_specs=[
            pl.BlockSpec((NC, _BLK, D), lambda i: (0, i, 0)),
            pl.BlockSpec((NC, _BLK, 16), lambda i: (0, i, 0)),
            pl.BlockSpec((_BLK, D), lambda i: (i, 0)),
            pl.BlockSpec((16, D), lambda i: (0, 0)),
            pl.BlockSpec((D, D), lambda i: (0, 0)),
            pl.BlockSpec((1, D), lambda i: (0, 0)),
            pl.BlockSpec((1, D), lambda i: (0, 0)),
            pl.BlockSpec((1, D), lambda i: (0, 0)),
            pl.BlockSpec((D, FF), lambda i: (0, 0)),
            pl.BlockSpec((1, FF), lambda i: (0, 0)),
            pl.BlockSpec((FF, D), lambda i: (0, 0)),
            pl.BlockSpec((1, D), lambda i: (0, 0)),
            pl.BlockSpec((1, D), lambda i: (0, 0)),
            pl.BlockSpec((1, D), lambda i: (0, 0)),
        ],
        out_specs=pl.BlockSpec((_BLK, D), lambda i: (i, 0)),
        out_shape=jax.ShapeDtypeStruct((N, D), jnp.float32),
    )(awv, az, x, e2, wo, bo, g1, b1n, w1, b1, w2, b2, g2, b2n)


_E2 = np.zeros((16, D), np.float32)
for _h in range(H):
    _E2[_h, _h * DK:(_h + 1) * DK] = 1.0


def kernel(x, edge_index, edges, rel_embed, Wq, bq, Wk, Wv, Wo, bo,
           ln1_g, ln1_b, W1, b1, W2, b2, ln2_g, ln2_b):
    src = edge_index[0]
    dst = edge_index[1]
    e2 = jnp.asarray(_E2)
    for i in range(L):
        wstack = jnp.stack([Wq[i], Wk[i], Wv[i]])
        q, k, v = _tc_qkv(x, wstack, bq[i].reshape(1, D))
        awv, az = _sc_edge(q, k, v, rel_embed, src, dst, edges)
        x = _tc_post(awv, az, x, e2,
                     Wo[i], bo[i].reshape(1, D),
                     ln1_g[i].reshape(1, D), ln1_b[i].reshape(1, D),
                     W1[i], b1[i].reshape(1, FF),
                     W2[i], b2[i].reshape(1, D),
                     ln2_g[i].reshape(1, D), ln2_b[i].reshape(1, D))
    return x


# speedup vs baseline: 47.3393x; 1.8621x over previous
import functools
import numpy as np
import jax
import jax.numpy as jnp
from jax import lax
from jax.experimental import pallas as pl
from jax.experimental.pallas import tpu as pltpu
from jax.experimental.pallas import tpu_sc as plsc

N = 10000
E = 320000
D = 128
H = 8
DK = 16
L = 2
R = 50
FF = 4 * D

NC = 2
NS = 16
NW = NC * NS
EPW = E // NW
C = 40
NCHUNK = EPW // C
RPT = 624
REM = N - NS * RPT
ZR = 48


_BLK = 1000


def _qkv_body(x_ref, w_ref, bq_ref, q_ref, k_ref, v_ref):
    x = x_ref[...]
    q_ref[...] = jnp.dot(x, w_ref[0], preferred_element_type=jnp.float32) + bq_ref[...]
    k_ref[...] = jnp.dot(x, w_ref[1], preferred_element_type=jnp.float32)
    v_ref[...] = jnp.dot(x, w_ref[2], preferred_element_type=jnp.float32)


def _tc_qkv(x, wstack, bq):
    return pl.pallas_call(
        _qkv_body,
        grid=(N // _BLK,),
        in_specs=[
            pl.BlockSpec((_BLK, D), lambda i: (i, 0)),
            pl.BlockSpec((3, D, D), lambda i: (0, 0, 0)),
            pl.BlockSpec((1, D), lambda i: (0, 0)),
        ],
        out_specs=[
            pl.BlockSpec((_BLK, D), lambda i: (i, 0)),
            pl.BlockSpec((_BLK, D), lambda i: (i, 0)),
            pl.BlockSpec((_BLK, D), lambda i: (i, 0)),
        ],
        out_shape=[
            jax.ShapeDtypeStruct((N, D), jnp.float32),
            jax.ShapeDtypeStruct((N, D), jnp.float32),
            jax.ShapeDtypeStruct((N, D), jnp.float32),
        ],
    )(x, wstack, bq)


def _sc_edge_body(q_hbm, k_hbm, v_hbm, rel_hbm, src_hbm, dst_hbm, rid_hbm,
                  zwv_hbm, zz_hbm,
                  outwv_hbm, outz_hbm,
                  accwv, accz,
                  sidx0, sidx1, didx0, didx1, ridx0, ridx1,
                  kr0, kr1, qr0, qr1, vr0, vr1, re0, re1,
                  msg, zbuf, sem0, sem1):
    c = lax.axis_index("c")
    s = lax.axis_index("s")
    wid = c * NS + s

    bufs = ((sidx0, didx0, ridx0, kr0, qr0, vr0, re0, sem0),
            (sidx1, didx1, ridx1, kr1, qr1, vr1, re1, sem1))

    zero16 = jnp.zeros((16,), jnp.float32)

    row0 = s * RPT
    pltpu.sync_copy(zwv_hbm.at[pl.ds(row0, RPT)], accwv.at[pl.ds(row0, RPT)])
    pltpu.sync_copy(zz_hbm.at[pl.ds(row0, RPT)], accz.at[pl.ds(row0, RPT)])

    @pl.when(s == NS - 1)
    def _zero_tail():
        pltpu.sync_copy(zwv_hbm.at[pl.ds(NS * RPT, REM)],
                        accwv.at[pl.ds(NS * RPT, REM)])
        pltpu.sync_copy(zz_hbm.at[pl.ds(NS * RPT, REM)],
                        accz.at[pl.ds(NS * RPT, REM)])

    plsc.subcore_barrier()

    iota = lax.iota(jnp.int32, 16)
    inv_scale = 1.0 / float(np.sqrt(DK))
    shuf = [(iota + sh) & 15 for sh in (8, 4, 2, 1)]

    dnums = lax.GatherDimensionNumbers(
        offset_dims=(), collapsed_slice_dims=(0,), start_index_map=(0,))

    def _allsum(t):
        for sx in shuf:
            t = t + lax.gather(t, sx[:, None], dnums, slice_sizes=(1,),
                               mode=lax.GatherScatterMode.PROMISE_IN_BOUNDS)
        return t

    def _load_idx(g, buf):
        si, di, ri = buf[0], buf[1], buf[2]
        ebase = wid * EPW + g * C
        pltpu.sync_copy(src_hbm.at[pl.ds(ebase, C)], si)
        pltpu.sync_copy(dst_hbm.at[pl.ds(ebase, C)], di)
        pltpu.sync_copy(rid_hbm.at[pl.ds(ebase, C)], ri)

    def _gather_copies(buf):
        si, di, ri, kr, qr, vr, re, sem = buf
        return (
            pltpu.make_async_copy(k_hbm.at[si], kr, sem),
            pltpu.make_async_copy(q_hbm.at[di], qr, sem),
            pltpu.make_async_copy(v_hbm.at[si], vr, sem),
            pltpu.make_async_copy(rel_hbm.at[ri], re, sem),
        )

    def _start_gathers(buf):
        for cp in _gather_copies(buf):
            cp.start()

    def _wait_gathers(buf):
        for cp in _gather_copies(buf):
            cp.wait()

    def _compute_chunk(buf):
        si, di, ri, kr, qr, vr, re, sem = buf

        def _edge(i, ecarry):
            rel = re[i, :]
            zv = zero16
            for h in range(H):
                sl = pl.ds(h * 16, 16)
                kh = kr[i, sl]
                qh = qr[i, sl]
                vh = vr[i, sl]
                sb = _allsum((kh + rel) * qh) * inv_scale
                se = jnp.exp(jnp.clip(sb, -10.0, 10.0))
                msg[i, sl] = (vh + rel) * se
                zv = jnp.where(iota == h, se, zv)
            zbuf[i, :] = zv
            return ecarry

        lax.fori_loop(0, C, _edge, 0)
        pltpu.sync_copy(msg, accwv.at[di], add=True)
        pltpu.sync_copy(zbuf, accz.at[di], add=True)

    _load_idx(0, bufs[0])
    _start_gathers(bufs[0])

    def _step(t2, carry):
        for b in (0, 1):
            g = t2 * 2 + b
            cur = bufs[b]
            nxt = bufs[1 - b]

            @pl.when(g < NCHUNK - 1)
            def _prefetch():
                _load_idx(g + 1, nxt)
                _start_gathers(nxt)

            _wait_gathers(cur)
            _compute_chunk(cur)
        return carry

    lax.fori_loop(0, NCHUNK // 2, _step, 0)
    plsc.subcore_barrier()

    row0 = s * RPT
    pltpu.sync_copy(accwv.at[pl.ds(row0, RPT)], outwv_hbm.at[c, pl.ds(row0, RPT)])
    pltpu.sync_copy(accz.at[pl.ds(row0, RPT)], outz_hbm.at[c, pl.ds(row0, RPT)])

    @pl.when(s == NS - 1)
    def _copy_tail():
        pltpu.sync_copy(accwv.at[pl.ds(NS * RPT, REM)],
                        outwv_hbm.at[c, pl.ds(NS * RPT, REM)])
        pltpu.sync_copy(accz.at[pl.ds(NS * RPT, REM)],
                        outz_hbm.at[c, pl.ds(NS * RPT, REM)])


_sc_edge = functools.partial(
    pl.kernel,
    _sc_edge_body,
    out_type=(
        jax.ShapeDtypeStruct((NC, N, D), jnp.float32),
        jax.ShapeDtypeStruct((NC, N, 16), jnp.float32),
    ),
    mesh=plsc.VectorSubcoreMesh(core_axis_name="c", subcore_axis_name="s"),
    compiler_params=pltpu.CompilerParams(use_tc_tiling_on_sc=False),
    scratch_types=(
        [
            pltpu.VMEM_SHARED((N, D), jnp.float32),
            pltpu.VMEM_SHARED((N, 16), jnp.float32),
        ]
        + [pltpu.VMEM((C,), jnp.int32)] * 6
        + [pltpu.VMEM((C, D), jnp.float32)] * 6
        + [pltpu.VMEM((C, 16), jnp.float32)] * 2
        + [
            pltpu.VMEM((C, D), jnp.float32),
            pltpu.VMEM((C, 16), jnp.float32),
            pltpu.SemaphoreType.DMA,
            pltpu.SemaphoreType.DMA,
        ]
    ),
)()


def _ln(a, g, b, eps=1e-5):
    m = jnp.mean(a, axis=-1, keepdims=True)
    v = jnp.mean((a - m) ** 2, axis=-1, keepdims=True)
    return g * (a - m) / jnp.sqrt(v + eps) + b


def _post_body(awv_ref, az_ref, x_ref, e2_ref, wo_ref, bo_ref, g1_ref, b1n_ref,
               w1_ref, b1_ref, w2_ref, b2_ref, g2_ref, b2n_ref, out_ref):
    wv = awv_ref[0] + awv_ref[1]
    zh = az_ref[0] + az_ref[1]
    z128 = jnp.dot(zh, e2_ref[...], preferred_element_type=jnp.float32)
    o = wv / (z128 + 1e-12)
    a = x_ref[...] + jnp.dot(o, wo_ref[...], preferred_element_type=jnp.float32) + bo_ref[...]
    x1 = _ln(a, g1_ref[...], b1n_ref[...])
    hmid = jnp.maximum(jnp.dot(x1, w1_ref[...], preferred_element_type=jnp.float32) + b1_ref[...], 0.0)
    hout = jnp.dot(hmid, w2_ref[...], preferred_element_type=jnp.float32) + b2_ref[...]
    out_ref[...] = _ln(x1 + hout, g2_ref[...], b2n_ref[...])


def _tc_post(awv, az, x, e2, wo, bo, g1, b1n, w1, b1, w2, b2, g2, b2n):
    return pl.pallas_call(
        _post_body,
        grid=(N // _BLK,),
        in_specs=[
            pl.BlockSpec((NC, _BLK, D), lambda i: (0, i, 0)),
            pl.BlockSpec((NC, _BLK, 16), lambda i: (0, i, 0)),
            pl.BlockSpec((_BLK, D), lambda i: (i, 0)),
            pl.BlockSpec((16, D), lambda i: (0, 0)),
            pl.BlockSpec((D, D), lambda i: (0, 0)),
            pl.BlockSpec((1, D), lambda i: (0, 0)),
            pl.BlockSpec((1, D), lambda i: (0, 0)),
            pl.BlockSpec((1, D), lambda i: (0, 0)),
            pl.BlockSpec((D, FF), lambda i: (0, 0)),
            pl.BlockSpec((1, FF), lambda i: (0, 0)),
            pl.BlockSpec((FF, D), lambda i: (0, 0)),
            pl.BlockSpec((1, D), lambda i: (0, 0)),
            pl.BlockSpec((1, D), lambda i: (0, 0)),
            pl.BlockSpec((1, D), lambda i: (0, 0)),
        ],
        out_specs=pl.BlockSpec((_BLK, D), lambda i: (i, 0)),
        out_shape=jax.ShapeDtypeStruct((N, D), jnp.float32),
    )(awv, az, x, e2, wo, bo, g1, b1n, w1, b1, w2, b2, g2, b2n)


_E2 = np.zeros((16, D), np.float32)
for _h in range(H):
    _E2[_h, _h * DK:(_h + 1) * DK] = 1.0


def kernel(x, edge_index, edges, rel_embed, Wq, bq, Wk, Wv, Wo, bo,
           ln1_g, ln1_b, W1, b1, W2, b2, ln2_g, ln2_b):
    src = edge_index[0]
    dst = edge_index[1]
    e2 = jnp.asarray(_E2)
    zwv = jnp.zeros((N, D), jnp.float32)
    zz = jnp.zeros((N, 16), jnp.float32)
    for i in range(L):
        wstack = jnp.stack([Wq[i], Wk[i], Wv[i]])
        q, k, v = _tc_qkv(x, wstack, bq[i].reshape(1, D))
        awv, az = _sc_edge(q, k, v, rel_embed, src, dst, edges, zwv, zz)
        x = _tc_post(awv, az, x, e2,
                     Wo[i], bo[i].reshape(1, D),
                     ln1_g[i].reshape(1, D), ln1_b[i].reshape(1, D),
                     W1[i], b1[i].reshape(1, FF),
                     W2[i], b2[i].reshape(1, D),
                     ln2_g[i].reshape(1, D), ln2_b[i].reshape(1, D))
    return x
